# Initial kernel scaffold; baseline (speedup 1.0000x reference)
import jax
import jax.numpy as jnp
from jax.experimental import pallas as pl


def kernel(query, reference_points, input_flatten, input_spatial_shapes, input_level_start_index, W_off, b_off, W_attn, b_attn, W_val, b_val, W_out, b_out):
    raise NotImplementedError("write your pallas kernel here")


# speedup vs baseline: 34.5851x; 34.5851x over previous
import functools

import jax
import jax.numpy as jnp
from jax import lax
from jax.experimental import pallas as pl
from jax.experimental.pallas import tpu as pltpu
from jax.experimental.pallas import tpu_sc as plsc

_N = 2
_LQ = 5440
_C = 256
_M = 8
_L = 4
_P = 4
_LEN = 5440
_QB = 1088
_QC = 320
_GPC = _QC // 16
_NCHUNK = _LQ // _QC

_SIZES = (64, 32, 16, 8)
_STARTS = (0, 4096, 5120, 5376)


def _stage1(q_ref, x_ref, rp_ref, woff_ref, boff_ref, wattn_ref, battn_ref,
            wval_ref, bval_ref, val_ref, pxy_ref, aw_ref):
    q = q_ref[0]
    x = x_ref[0]
    val_ref[0] = (jnp.dot(x, wval_ref[...], preferred_element_type=jnp.float32)
                  + bval_ref[...])

    off = (jnp.dot(q, woff_ref[...], preferred_element_type=jnp.float32)
           + boff_ref[...])
    rp = rp_ref[0]
    rpb = jnp.broadcast_to(rp[:, None, :, None, :], (_QB, _M, _L, _P, 2))
    rpb = rpb.reshape(_QB, 256)
    cidx = lax.broadcasted_iota(jnp.int32, (1, 256), 1)
    lvl = (cidx // (2 * _P)) % _L
    wl = jnp.left_shift(1, 6 - lvl).astype(jnp.float32)
    pxy_ref[0] = (rpb + off) * wl - 0.5

    a = (jnp.dot(q, wattn_ref[...], preferred_element_type=jnp.float32)
         + battn_ref[...])
    a3 = a.reshape(_QB, _M, _L * _P)
    amax = jnp.max(a3, axis=2, keepdims=True)
    e = jnp.exp(a3 - amax)
    s = jnp.sum(e, axis=2, keepdims=True)
    aw_ref[0] = (e / s).reshape(_QB, 128)


def _stage3(attn_ref, wout_ref, bout_ref, out_ref):
    a = attn_ref[0]
    out_ref[0] = (jnp.dot(a, wout_ref[...], preferred_element_type=jnp.float32)
                  + bout_ref[...])


def _sc_body(val_hbm, pxy_hbm, aw_hbm, out_hbm, val_v, pxy_v, aw_v, out_v):
    cid = lax.axis_index("c")
    sid = lax.axis_index("s")
    wid = sid * 2 + cid
    n = wid // 16
    mh = wid - n * 16
    m = mh // 2

    pltpu.sync_copy(val_hbm.at[n, :, mh, :], val_v)

    iota16 = lax.iota(jnp.int32, 16)
    csplat = [jnp.full((16,), c, jnp.int32) for c in range(32)]

    def chunk_body(ci, carry):
        q0 = ci * _QC
        pltpu.sync_copy(pxy_hbm.at[n, pl.ds(q0, _QC), pl.ds(m * 32, 32)],
                        pxy_v)
        pltpu.sync_copy(aw_hbm.at[n, pl.ds(q0, _QC), pl.ds(m * 16, 16)],
                        aw_v)

        def group_body(g, gcarry):
            qi = iota16 + g * 16
            accs = [jnp.zeros((16,), jnp.float32)] * 16
            for lp in range(16):
                wi = _SIZES[lp // _P]
                base = _STARTS[lp // _P]
                xx = plsc.load_gather(pxy_v, [qi, csplat[2 * lp]])
                yy = plsc.load_gather(pxy_v, [qi, csplat[2 * lp + 1]])
                aa = plsc.load_gather(aw_v, [qi, csplat[lp]])

                xt = xx.astype(jnp.int32)
                xf = xt.astype(jnp.float32)
                ix0 = xt - (xf > xx).astype(jnp.int32)
                fx0 = ix0.astype(jnp.float32)
                wx1 = xx - fx0
                wx0 = 1.0 - wx1

                yt = yy.astype(jnp.int32)
                yf = yt.astype(jnp.float32)
                iy0 = yt - (yf > yy).astype(jnp.int32)
                fy0 = iy0.astype(jnp.float32)
                wy1 = yy - fy0
                wy0 = 1.0 - wy1

                zero = jnp.zeros((16,), jnp.float32)
                wx0 = jnp.where((ix0 >= 0) & (ix0 <= wi - 1), wx0, zero)
                wx1 = jnp.where((ix0 >= -1) & (ix0 <= wi - 2), wx1, zero)
                wy0 = jnp.where((iy0 >= 0) & (iy0 <= wi - 1), wy0, zero)
                wy1 = jnp.where((iy0 >= -1) & (iy0 <= wi - 2), wy1, zero)

                ix0c = jnp.clip(ix0, 0, wi - 1)
                ix1c = jnp.clip(ix0 + 1, 0, wi - 1)
                iy0c = jnp.clip(iy0, 0, wi - 1)
                iy1c = jnp.clip(iy0 + 1, 0, wi - 1)

                ax0 = wx0 * aa
                ax1 = wx1 * aa
                w00 = ax0 * wy0
                w01 = ax1 * wy0
                w10 = ax0 * wy1
                w11 = ax1 * wy1

                t0 = iy0c * wi + base
                t1 = iy1c * wi + base
                r00 = t0 + ix0c
                r01 = t0 + ix1c
                r10 = t1 + ix0c
                r11 = t1 + ix1c

                for c in range(16):
                    g00 = plsc.load_gather(val_v, [r00, csplat[c]])
                    g01 = plsc.load_gather(val_v, [r01, csplat[c]])
                    g10 = plsc.load_gather(val_v, [r10, csplat[c]])
                    g11 = plsc.load_gather(val_v, [r11, csplat[c]])
                    accs[c] = (accs[c] + g00 * w00 + g01 * w01
                               + g10 * w10 + g11 * w11)
            for c in range(16):
                plsc.store_scatter(out_v, [qi, csplat[c]], accs[c])
            return gcarry

        lax.fori_loop(0, _GPC, group_body, 0)
        pltpu.sync_copy(out_v, out_hbm.at[n, pl.ds(q0, _QC), mh, :])
        return carry

    lax.fori_loop(0, _NCHUNK, chunk_body, 0)


def _sc_sample(val_r, pxy, aw):
    mesh = plsc.VectorSubcoreMesh(core_axis_name="c", subcore_axis_name="s")
    f = functools.partial(
        pl.kernel,
        out_type=jax.ShapeDtypeStruct((_N, _LQ, 16, 16), jnp.float32),
        mesh=mesh,
        scratch_types=[
            pltpu.VMEM((_LEN, 16), jnp.float32),
            pltpu.VMEM((_QC, 32), jnp.float32),
            pltpu.VMEM((_QC, 16), jnp.float32),
            pltpu.VMEM((_QC, 16), jnp.float32),
        ],
        compiler_params=pltpu.CompilerParams(use_tc_tiling_on_sc=False,
                                             needs_layout_passes=False),
    )(_sc_body)
    return f(val_r, pxy, aw)


def kernel(query, reference_points, input_flatten, input_spatial_shapes,
           input_level_start_index, W_off, b_off, W_attn, b_attn,
           W_val, b_val, W_out, b_out):
    del input_spatial_shapes, input_level_start_index
    grid = (_N, _LQ // _QB)

    val, pxy, aw = pl.pallas_call(
        _stage1,
        grid=grid,
        in_specs=[
            pl.BlockSpec((1, _QB, _C), lambda n, i: (n, i, 0)),
            pl.BlockSpec((1, _QB, _C), lambda n, i: (n, i, 0)),
            pl.BlockSpec((1, _QB, _L, 2), lambda n, i: (n, i, 0, 0)),
            pl.BlockSpec((_C, 256), lambda n, i: (0, 0)),
            pl.BlockSpec((1, 256), lambda n, i: (0, 0)),
            pl.BlockSpec((_C, 128), lambda n, i: (0, 0)),
            pl.BlockSpec((1, 128), lambda n, i: (0, 0)),
            pl.BlockSpec((_C, _C), lambda n, i: (0, 0)),
            pl.BlockSpec((1, _C), lambda n, i: (0, 0)),
        ],
        out_specs=[
            pl.BlockSpec((1, _QB, _C), lambda n, i: (n, i, 0)),
            pl.BlockSpec((1, _QB, 256), lambda n, i: (n, i, 0)),
            pl.BlockSpec((1, _QB, 128), lambda n, i: (n, i, 0)),
        ],
        out_shape=[
            jax.ShapeDtypeStruct((_N, _LQ, _C), jnp.float32),
            jax.ShapeDtypeStruct((_N, _LQ, 256), jnp.float32),
            jax.ShapeDtypeStruct((_N, _LQ, 128), jnp.float32),
        ],
    )(query, input_flatten, reference_points,
      W_off, b_off.reshape(1, -1), W_attn, b_attn.reshape(1, -1),
      W_val, b_val.reshape(1, -1))

    val_r = val.reshape(_N, _LEN, 16, 16)
    attn = _sc_sample(val_r, pxy, aw)
    attn_r = attn.reshape(_N, _LQ, _C)

    out = pl.pallas_call(
        _stage3,
        grid=grid,
        in_specs=[
            pl.BlockSpec((1, _QB, _C), lambda n, i: (n, i, 0)),
            pl.BlockSpec((_C, _C), lambda n, i: (0, 0)),
            pl.BlockSpec((1, _C), lambda n, i: (0, 0)),
        ],
        out_specs=pl.BlockSpec((1, _QB, _C), lambda n, i: (n, i, 0)),
        out_shape=jax.ShapeDtypeStruct((_N, _LQ, _C), jnp.float32),
    )(attn_r, W_out, b_out.reshape(1, -1))
    return out
